# baseline (speedup 1.0000x reference)
import functools

import jax
import jax.numpy as jnp
from jax import lax
from jax.experimental import pallas as pl
from jax.experimental.pallas import tpu as pltpu
from jax.experimental.pallas import tpu_sc as plsc

_N = 2048
_B = 8
_E = 12288
_NTILES = 32
_CHUNK_ROWS = 32
_LANES = 16

_DIMS = [1536, 512, 512, 256, 256, 128, 128]


def _adj_body(edges_hbm, out_hbm, src_v, dst_v, block_v):
    wid = lax.axis_index("s") * 2 + lax.axis_index("c")
    pltpu.sync_copy(edges_hbm.at[0], src_v)
    pltpu.sync_copy(edges_hbm.at[1], dst_v)

    big = jnp.int32(2147483647)
    pos = lax.iota(jnp.int32, _LANES)
    ones = jnp.ones((_LANES,), jnp.float32)

    for c in range(2):
        base = (wid * 2 + c) * _CHUNK_ROWS

        def zero_body(i, carry):
            block_v[i >> 7, pl.ds((i & 127) * _LANES, _LANES)] = (
                jnp.zeros((_LANES,), jnp.float32))
            return carry

        lax.fori_loop(0, _CHUNK_ROWS * _N // _LANES, zero_body, 0, unroll=8)

        def edge_body(i, carry):
            d = dst_v[pl.ds(i * _LANES, _LANES)]
            s = src_v[pl.ds(i * _LANES, _LANES)]
            valid = (d >= base) & (d < base + _CHUNK_ROWS)

            @pl.when(jnp.any(valid))
            def _():
                local = (d - base) * _N + s
                key = jnp.where(valid, local, big)
                skey = jnp.sort(key)
                prev = skey.at[jnp.maximum(pos - 1, 0)].get(
                    mode="promise_in_bounds")
                nxt = skey.at[jnp.minimum(pos + 1, _LANES - 1)].get(
                    mode="promise_in_bounds")
                is_start = (skey != prev) | (pos == 0)
                is_end = ((skey != nxt) | (pos == _LANES - 1)) & (skey != big)
                run_start = jnp.where(is_start, pos, 0)
                for sh in (1, 2, 4, 8):
                    shifted = run_start.at[jnp.maximum(pos - sh, 0)].get(
                        mode="promise_in_bounds")
                    run_start = jnp.maximum(
                        run_start, jnp.where(pos >= sh, shifted, 0))
                cnt = (pos - run_start + 1).astype(jnp.float32)
                rows = jnp.where(is_end, lax.shift_right_logical(skey, 11), 0)
                cols = jnp.where(is_end, jnp.bitwise_and(skey, _N - 1), 0)
                plsc.addupdate_scatter(block_v, [rows, cols], cnt,
                                       mask=is_end)

            return carry

        lax.fori_loop(0, _E // _LANES, edge_body, 0)

        for j in range(2):
            r = pos + j * _LANES
            plsc.addupdate_scatter(block_v, [r, base + r], ones)

        pltpu.sync_copy(block_v, out_hbm.at[pl.ds(base, _CHUNK_ROWS)])


def _build_adj(edges):
    k = pl.kernel(
        _adj_body,
        out_type=jax.ShapeDtypeStruct((_N, _N), jnp.float32),
        mesh=plsc.VectorSubcoreMesh(core_axis_name="c", subcore_axis_name="s"),
        compiler_params=pltpu.CompilerParams(needs_layout_passes=False,
                                             use_tc_tiling_on_sc=True),
        scratch_types=[
            pltpu.VMEM((_E,), jnp.int32),
            pltpu.VMEM((_E,), jnp.int32),
            pltpu.VMEM((_CHUNK_ROWS, _N), jnp.float32),
        ],
    )
    return k(edges)


def _deg_kernel(adj_ref, o_ref):
    deg = jnp.sum(adj_ref[...], axis=1)
    o_ref[0, 0, :] = lax.rsqrt(deg)


def _compute_dinv(adj):
    out = pl.pallas_call(
        _deg_kernel,
        grid=(_N // 128,),
        in_specs=[pl.BlockSpec((128, _N), lambda i: (i, 0))],
        out_specs=pl.BlockSpec((1, 1, 128), lambda i: (i, 0, 0)),
        out_shape=jax.ShapeDtypeStruct((_N // 128, 1, 128), jnp.float32),
    )(adj)
    return out.reshape(_N)


def _scale_kernel(adj_ref, dcol_ref, drow_ref, o_ref):
    v = adj_ref[...] * dcol_ref[:, 0:1] * drow_ref[...]
    o_ref[...] = v.astype(jnp.bfloat16)


def _normalize_adj(adj, dinv):
    dcol = jnp.broadcast_to(dinv.reshape(_N, 1), (_N, 128))
    drow = dinv.reshape(1, _N)
    return pl.pallas_call(
        _scale_kernel,
        grid=(_N // 128,),
        in_specs=[
            pl.BlockSpec((128, _N), lambda i: (i, 0)),
            pl.BlockSpec((128, 128), lambda i: (i, 0)),
            pl.BlockSpec((1, _N), lambda i: (0, 0)),
        ],
        out_specs=pl.BlockSpec((128, _N), lambda i: (i, 0)),
        out_shape=jax.ShapeDtypeStruct((_N, _N), jnp.bfloat16),
    )(adj, dcol, drow)


def _xw_kernel(x_ref, w_ref, o_ref):
    v = jnp.dot(x_ref[...], w_ref[...], preferred_element_type=jnp.float32)
    o_ref[...] = v.astype(jnp.bfloat16)


def _dense(x, w, bm=1024):
    m, kdim = x.shape
    _, n = w.shape
    return pl.pallas_call(
        _xw_kernel,
        grid=(m // bm,),
        in_specs=[
            pl.BlockSpec((bm, kdim), lambda i: (i, 0)),
            pl.BlockSpec((kdim, n), lambda i: (0, 0)),
        ],
        out_specs=pl.BlockSpec((bm, n), lambda i: (i, 0)),
        out_shape=jax.ShapeDtypeStruct((m, n), jnp.bfloat16),
        compiler_params=pltpu.CompilerParams(
            dimension_semantics=("arbitrary",)),
    )(x, w)


def _gm_kernel(g_ref, w_ref, o_ref):
    v = jnp.dot(g_ref[...].astype(jnp.bfloat16), w_ref[...],
                preferred_element_type=jnp.float32)
    o_ref[...] = v


def _global_matmul(gf, w0c):
    return pl.pallas_call(
        _gm_kernel,
        out_shape=jax.ShapeDtypeStruct((_B, w0c.shape[1]), jnp.float32),
    )(gf, w0c)


def _l0_kernel(lf_ref, bv_ref, gm_ref, wb_ref, wa_ref, o_ref):
    x = lf_ref[...].astype(jnp.bfloat16)
    v = jnp.dot(x, wb_ref[...], preferred_element_type=jnp.float32)
    bv = bv_ref[...]
    for c in range(3):
        v += bv[:, c:c + 1] * wa_ref[c:c + 1, :]
    v += gm_ref[0]
    o_ref[...] = v.astype(jnp.bfloat16)


def _layer0(lf, bv, gm, w0b, w0a, bm=1024):
    m, kdim = lf.shape
    n = w0b.shape[1]
    return pl.pallas_call(
        _l0_kernel,
        grid=(m // bm,),
        in_specs=[
            pl.BlockSpec((bm, kdim), lambda i: (i, 0)),
            pl.BlockSpec((bm, 3), lambda i: (i, 0)),
            pl.BlockSpec((1, 1, n), lambda i: (i * bm // _N, 0, 0)),
            pl.BlockSpec((kdim, n), lambda i: (0, 0)),
            pl.BlockSpec((3, n), lambda i: (0, 0)),
        ],
        out_specs=pl.BlockSpec((bm, n), lambda i: (i, 0)),
        out_shape=jax.ShapeDtypeStruct((m, n), jnp.bfloat16),
        compiler_params=pltpu.CompilerParams(
            dimension_semantics=("arbitrary",)),
    )(lf, bv, gm, w0b, w0a)


def _leaky(v):
    return jnp.where(v >= 0, v, 0.01 * v)


def _agg_kernel(a_ref, h_ref, b_ref, o_ref, *, act):
    v = jnp.dot(a_ref[...], h_ref[0], preferred_element_type=jnp.float32)
    v = v + b_ref[...]
    v = _leaky(v) if act else v
    o_ref[0] = v.astype(o_ref.dtype)


def _agg(a, h, b, act, out_dtype=jnp.bfloat16):
    n = h.shape[-1]
    return pl.pallas_call(
        functools.partial(_agg_kernel, act=act),
        grid=(_B,),
        in_specs=[
            pl.BlockSpec((_N, _N), lambda i: (0, 0)),
            pl.BlockSpec((1, _N, n), lambda i: (i, 0, 0)),
            pl.BlockSpec((1, n), lambda i: (0, 0)),
        ],
        out_specs=pl.BlockSpec((1, _N, n), lambda i: (i, 0, 0)),
        out_shape=jax.ShapeDtypeStruct((_B, _N, n), out_dtype),
        compiler_params=pltpu.CompilerParams(
            dimension_semantics=("arbitrary",)),
    )(a, h, b)


def _dec_kernel(f_ref, w_ref, b_ref, o_ref):
    v = jnp.dot(f_ref[...], w_ref[...], preferred_element_type=jnp.float32)
    o_ref[...] = jnp.tanh(v + b_ref[...]) * 0.1


def _decoder(feats, w_dec, b_dec, bn=512):
    m, kdim = feats.shape
    return pl.pallas_call(
        _dec_kernel,
        grid=(kdim // bn,),
        in_specs=[
            pl.BlockSpec((m, kdim), lambda i: (0, 0)),
            pl.BlockSpec((kdim, bn), lambda i: (0, i)),
            pl.BlockSpec((1, bn), lambda i: (0, i)),
        ],
        out_specs=pl.BlockSpec((m, bn), lambda i: (0, i)),
        out_shape=jax.ShapeDtypeStruct((m, kdim), jnp.float32),
        compiler_params=pltpu.CompilerParams(
            dimension_semantics=("arbitrary",)),
    )(feats, w_dec, b_dec)


def _pad2(w, rows, cols):
    r, c = w.shape
    return jnp.pad(w, ((0, rows - r), (0, cols - c)))


def kernel(batch_vertices, local_features, global_features, edges,
           W0, b0, W1, b1, W2, b2, W3, b3, W4, b4, W5, b5, W_dec, b_dec):
    adj = _build_adj(edges)
    dinv = _compute_dinv(adj)
    a = _normalize_adj(adj, dinv)

    nf = local_features.shape[-1]
    w0a = W0[:3]
    w0b = W0[3:3 + nf].astype(jnp.bfloat16)
    w0c = W0[3 + nf:].astype(jnp.bfloat16)
    gm = _global_matmul(global_features, w0c).reshape(_B, 1, _DIMS[1])
    h = _layer0(local_features.reshape(_B * _N, nf),
                batch_vertices.reshape(_B * _N, 3), gm, w0b, w0a)
    x = _agg(a, h.reshape(_B, _N, _DIMS[1]), b0.reshape(1, -1), act=False)

    ws = [W1, W2, W3, W4, W5]
    bs = [b1, b2, b3, b4, b5]
    for i in range(1, 6):
        kin, kout = _DIMS[i], _DIMS[i + 1]
        w = _pad2(ws[i - 1], kin, kout).astype(jnp.bfloat16)
        b = jnp.pad(bs[i - 1],
                    (0, kout - bs[i - 1].shape[0])).reshape(1, kout)
        h = _dense(x.reshape(_B * _N, kin), w)
        x = _agg(a, h.reshape(_B, _N, kout), b, act=(i % 2 == 1),
                 out_dtype=(jnp.float32 if i == 5 else jnp.bfloat16))

    feats = x[..., :3].reshape(_B, _N * 3)
    out = _decoder(feats, W_dec, b_dec.reshape(1, _N * 3))
    return out.reshape(_B, _N, 3)


# speedup vs baseline: 1.1219x; 1.1219x over previous
import functools

import jax
import jax.numpy as jnp
from jax import lax
from jax.experimental import pallas as pl
from jax.experimental.pallas import tpu as pltpu
from jax.experimental.pallas import tpu_sc as plsc

_N = 2048
_B = 8
_E = 12288
_NTILES = 32
_CHUNK_ROWS = 32
_LANES = 16

_DIMS = [1536, 512, 512, 256, 256, 128, 128]


def _adj_body(edges_hbm, out_hbm, src_v, dst_v, block_v):
    wid = lax.axis_index("s") * 2 + lax.axis_index("c")
    pltpu.sync_copy(edges_hbm.at[0], src_v)
    pltpu.sync_copy(edges_hbm.at[1], dst_v)

    big = jnp.int32(2147483647)
    pos = lax.iota(jnp.int32, _LANES)
    ones = jnp.ones((_LANES,), jnp.float32)

    for c in range(2):
        base = (wid * 2 + c) * _CHUNK_ROWS

        def zero_body(i, carry):
            block_v[i >> 7, pl.ds((i & 127) * _LANES, _LANES)] = (
                jnp.zeros((_LANES,), jnp.float32))
            return carry

        lax.fori_loop(0, _CHUNK_ROWS * _N // _LANES, zero_body, 0, unroll=8)

        def edge_body(i, carry):
            d = dst_v[pl.ds(i * _LANES, _LANES)]
            s = src_v[pl.ds(i * _LANES, _LANES)]
            valid = (d >= base) & (d < base + _CHUNK_ROWS)

            @pl.when(jnp.any(valid))
            def _():
                local = (d - base) * _N + s
                key = jnp.where(valid, local, big)
                skey = jnp.sort(key)
                prev = skey.at[jnp.maximum(pos - 1, 0)].get(
                    mode="promise_in_bounds")
                nxt = skey.at[jnp.minimum(pos + 1, _LANES - 1)].get(
                    mode="promise_in_bounds")
                is_start = (skey != prev) | (pos == 0)
                is_end = ((skey != nxt) | (pos == _LANES - 1)) & (skey != big)
                run_start = jnp.where(is_start, pos, 0)
                for sh in (1, 2, 4, 8):
                    shifted = run_start.at[jnp.maximum(pos - sh, 0)].get(
                        mode="promise_in_bounds")
                    run_start = jnp.maximum(
                        run_start, jnp.where(pos >= sh, shifted, 0))
                cnt = (pos - run_start + 1).astype(jnp.float32)
                rows = jnp.where(is_end, lax.shift_right_logical(skey, 11), 0)
                cols = jnp.where(is_end, jnp.bitwise_and(skey, _N - 1), 0)
                plsc.addupdate_scatter(block_v, [rows, cols], cnt,
                                       mask=is_end)

            return carry

        lax.fori_loop(0, _E // _LANES, edge_body, 0)

        for j in range(2):
            r = pos + j * _LANES
            plsc.addupdate_scatter(block_v, [r, base + r], ones)

        pltpu.sync_copy(block_v, out_hbm.at[pl.ds(base, _CHUNK_ROWS)])


def _build_adj(edges):
    k = pl.kernel(
        _adj_body,
        out_type=jax.ShapeDtypeStruct((_N, _N), jnp.float32),
        mesh=plsc.VectorSubcoreMesh(core_axis_name="c", subcore_axis_name="s"),
        compiler_params=pltpu.CompilerParams(needs_layout_passes=False,
                                             use_tc_tiling_on_sc=True),
        scratch_types=[
            pltpu.VMEM((_E,), jnp.int32),
            pltpu.VMEM((_E,), jnp.int32),
            pltpu.VMEM((_CHUNK_ROWS, _N), jnp.float32),
        ],
    )
    return k(edges)


def _deg_kernel(adj_ref, o_ref):
    deg = jnp.sum(adj_ref[...], axis=1)
    o_ref[0, 0, :] = lax.rsqrt(deg)


def _compute_dinv(adj):
    out = pl.pallas_call(
        _deg_kernel,
        grid=(_N // 128,),
        in_specs=[pl.BlockSpec((128, _N), lambda i: (i, 0))],
        out_specs=pl.BlockSpec((1, 1, 128), lambda i: (i, 0, 0)),
        out_shape=jax.ShapeDtypeStruct((_N // 128, 1, 128), jnp.float32),
    )(adj)
    return out.reshape(_N)


def _scale_kernel(adj_ref, dcol_ref, drow_ref, o_ref):
    v = adj_ref[...] * dcol_ref[:, 0:1] * drow_ref[...]
    o_ref[...] = v.astype(jnp.bfloat16)


def _normalize_adj(adj, dinv):
    dcol = jnp.broadcast_to(dinv.reshape(_N, 1), (_N, 128))
    drow = dinv.reshape(1, _N)
    return pl.pallas_call(
        _scale_kernel,
        grid=(_N // 128,),
        in_specs=[
            pl.BlockSpec((128, _N), lambda i: (i, 0)),
            pl.BlockSpec((128, 128), lambda i: (i, 0)),
            pl.BlockSpec((1, _N), lambda i: (0, 0)),
        ],
        out_specs=pl.BlockSpec((128, _N), lambda i: (i, 0)),
        out_shape=jax.ShapeDtypeStruct((_N, _N), jnp.bfloat16),
    )(adj, dcol, drow)


def _xw_kernel(x_ref, w_ref, o_ref):
    v = jnp.dot(x_ref[...], w_ref[...], preferred_element_type=jnp.float32)
    o_ref[...] = v.astype(jnp.bfloat16)


def _dense(x, w, bm=1024):
    m, kdim = x.shape
    _, n = w.shape
    return pl.pallas_call(
        _xw_kernel,
        grid=(m // bm,),
        in_specs=[
            pl.BlockSpec((bm, kdim), lambda i: (i, 0)),
            pl.BlockSpec((kdim, n), lambda i: (0, 0)),
        ],
        out_specs=pl.BlockSpec((bm, n), lambda i: (i, 0)),
        out_shape=jax.ShapeDtypeStruct((m, n), jnp.bfloat16),
        compiler_params=pltpu.CompilerParams(
            dimension_semantics=("arbitrary",)),
    )(x, w)


def _gm_kernel(g_ref, w_ref, o_ref):
    v = jnp.dot(g_ref[...].astype(jnp.bfloat16), w_ref[...],
                preferred_element_type=jnp.float32)
    o_ref[...] = v


def _global_matmul(gf, w0c):
    return pl.pallas_call(
        _gm_kernel,
        out_shape=jax.ShapeDtypeStruct((_B, w0c.shape[1]), jnp.float32),
    )(gf, w0c)


_DNT = (((0,), (0,)), ((), ()))


def _l0_kernel(lft_ref, bvt_ref, gm_ref, wb_ref, wa_ref, o_ref):
    xt = lft_ref[0].astype(jnp.bfloat16)
    v = lax.dot_general(xt, wb_ref[...], _DNT,
                        preferred_element_type=jnp.float32)
    v += lax.dot_general(bvt_ref[:, pl.program_id(0), :], wa_ref[...], _DNT,
                         preferred_element_type=jnp.float32)
    v += gm_ref[0]
    o_ref[0] = v.astype(jnp.bfloat16)


def _layer0(lft, bvt, gm, w0b, w0a, bm=512):
    kdim = lft.shape[1]
    n = w0b.shape[1]
    return pl.pallas_call(
        _l0_kernel,
        grid=(_B, _N // bm),
        in_specs=[
            pl.BlockSpec((1, kdim, bm), lambda b, m: (b, 0, m)),
            pl.BlockSpec((3, _B, bm), lambda b, m: (0, 0, m)),
            pl.BlockSpec((1, 1, n), lambda b, m: (b, 0, 0)),
            pl.BlockSpec((kdim, n), lambda b, m: (0, 0)),
            pl.BlockSpec((3, n), lambda b, m: (0, 0)),
        ],
        out_specs=pl.BlockSpec((1, bm, n), lambda b, m: (b, m, 0)),
        out_shape=jax.ShapeDtypeStruct((_B, _N, n), jnp.bfloat16),
        compiler_params=pltpu.CompilerParams(
            dimension_semantics=("arbitrary", "arbitrary"),
            fuse_transposed_lhs_in_matmul=True),
    )(lft, bvt, gm, w0b, w0a)


def _leaky(v):
    return jnp.where(v >= 0, v, 0.01 * v)


def _agg_kernel(a_ref, h_ref, b_ref, o_ref, *, act):
    v = jnp.dot(a_ref[...], h_ref[0], preferred_element_type=jnp.float32)
    v = v + b_ref[...]
    v = _leaky(v) if act else v
    o_ref[0] = v.astype(o_ref.dtype)


def _agg(a, h, b, act, out_dtype=jnp.bfloat16):
    n = h.shape[-1]
    return pl.pallas_call(
        functools.partial(_agg_kernel, act=act),
        grid=(_B,),
        in_specs=[
            pl.BlockSpec((_N, _N), lambda i: (0, 0)),
            pl.BlockSpec((1, _N, n), lambda i: (i, 0, 0)),
            pl.BlockSpec((1, n), lambda i: (0, 0)),
        ],
        out_specs=pl.BlockSpec((1, _N, n), lambda i: (i, 0, 0)),
        out_shape=jax.ShapeDtypeStruct((_B, _N, n), out_dtype),
        compiler_params=pltpu.CompilerParams(
            dimension_semantics=("arbitrary",)),
    )(a, h, b)


def _dec_kernel(f_ref, w_ref, b_ref, o_ref):
    v = jnp.dot(f_ref[...], w_ref[...], preferred_element_type=jnp.float32)
    o_ref[...] = jnp.tanh(v + b_ref[...]) * 0.1


def _decoder(feats, w_dec, b_dec, bn=512):
    m, kdim = feats.shape
    return pl.pallas_call(
        _dec_kernel,
        grid=(kdim // bn,),
        in_specs=[
            pl.BlockSpec((m, kdim), lambda i: (0, 0)),
            pl.BlockSpec((kdim, bn), lambda i: (0, i)),
            pl.BlockSpec((1, bn), lambda i: (0, i)),
        ],
        out_specs=pl.BlockSpec((m, bn), lambda i: (0, i)),
        out_shape=jax.ShapeDtypeStruct((m, kdim), jnp.float32),
        compiler_params=pltpu.CompilerParams(
            dimension_semantics=("arbitrary",)),
    )(feats, w_dec, b_dec)


def _pad2(w, rows, cols):
    r, c = w.shape
    return jnp.pad(w, ((0, rows - r), (0, cols - c)))


def kernel(batch_vertices, local_features, global_features, edges,
           W0, b0, W1, b1, W2, b2, W3, b3, W4, b4, W5, b5, W_dec, b_dec):
    adj = _build_adj(edges)
    dinv = _compute_dinv(adj)
    a = _normalize_adj(adj, dinv)

    nf = local_features.shape[-1]
    w0a = W0[:3]
    w0b = W0[3:3 + nf].astype(jnp.bfloat16)
    w0c = W0[3 + nf:].astype(jnp.bfloat16)
    gm = _global_matmul(global_features, w0c).reshape(_B, 1, _DIMS[1])
    lft = jnp.transpose(local_features, (0, 2, 1))
    bvt = jnp.transpose(batch_vertices, (2, 0, 1))
    h = _layer0(lft, bvt, gm, w0b, w0a)
    x = _agg(a, h, b0.reshape(1, -1), act=False)

    ws = [W1, W2, W3, W4, W5]
    bs = [b1, b2, b3, b4, b5]
    for i in range(1, 6):
        kin, kout = _DIMS[i], _DIMS[i + 1]
        w = _pad2(ws[i - 1], kin, kout).astype(jnp.bfloat16)
        b = jnp.pad(bs[i - 1],
                    (0, kout - bs[i - 1].shape[0])).reshape(1, kout)
        h = _dense(x.reshape(_B * _N, kin), w)
        x = _agg(a, h.reshape(_B, _N, kout), b, act=(i % 2 == 1),
                 out_dtype=(jnp.float32 if i == 5 else jnp.bfloat16))

    feats = x[..., :3].reshape(_B, _N * 3)
    out = _decoder(feats, W_dec, b_dec.reshape(1, _N * 3))
    return out.reshape(_B, _N, 3)


# speedup vs baseline: 1.5741x; 1.4030x over previous
import functools

import jax
import jax.numpy as jnp
from jax import lax
from jax.experimental import pallas as pl
from jax.experimental.pallas import tpu as pltpu
from jax.experimental.pallas import tpu_sc as plsc

_N = 2048
_B = 8
_E = 12288
_NTILES = 32
_CHUNK_ROWS = 32
_LANES = 16

_DIMS = [1536, 512, 512, 256, 256, 128, 128]


def _adj_body(edges_hbm, out_hbm, src_v, dst_v, block_v):
    wid = lax.axis_index("s") * 2 + lax.axis_index("c")
    pltpu.sync_copy(edges_hbm.at[0], src_v)
    pltpu.sync_copy(edges_hbm.at[1], dst_v)

    big = jnp.int32(2147483647)
    pos = lax.iota(jnp.int32, _LANES)
    ones = jnp.ones((_LANES,), jnp.float32)

    for c in range(2):
        base = (wid * 2 + c) * _CHUNK_ROWS

        def zero_body(i, carry):
            block_v[i >> 7, pl.ds((i & 127) * _LANES, _LANES)] = (
                jnp.zeros((_LANES,), jnp.float32))
            return carry

        lax.fori_loop(0, _CHUNK_ROWS * _N // _LANES, zero_body, 0, unroll=8)

        def edge_body(i, carry):
            d = dst_v[pl.ds(i * _LANES, _LANES)]
            s = src_v[pl.ds(i * _LANES, _LANES)]
            valid = (d >= base) & (d < base + _CHUNK_ROWS)

            @pl.when(jnp.any(valid))
            def _():
                local = (d - base) * _N + s
                key = jnp.where(valid, local, big)
                skey = jnp.sort(key)
                prev = skey.at[jnp.maximum(pos - 1, 0)].get(
                    mode="promise_in_bounds")
                nxt = skey.at[jnp.minimum(pos + 1, _LANES - 1)].get(
                    mode="promise_in_bounds")
                is_start = (skey != prev) | (pos == 0)
                is_end = ((skey != nxt) | (pos == _LANES - 1)) & (skey != big)
                run_start = jnp.where(is_start, pos, 0)
                for sh in (1, 2, 4, 8):
                    shifted = run_start.at[jnp.maximum(pos - sh, 0)].get(
                        mode="promise_in_bounds")
                    run_start = jnp.maximum(
                        run_start, jnp.where(pos >= sh, shifted, 0))
                cnt = (pos - run_start + 1).astype(jnp.float32)
                rows = jnp.where(is_end, lax.shift_right_logical(skey, 11), 0)
                cols = jnp.where(is_end, jnp.bitwise_and(skey, _N - 1), 0)
                plsc.addupdate_scatter(block_v, [rows, cols], cnt,
                                       mask=is_end)

            return carry

        lax.fori_loop(0, _E // _LANES, edge_body, 0)

        for j in range(2):
            r = pos + j * _LANES
            plsc.addupdate_scatter(block_v, [r, base + r], ones)

        pltpu.sync_copy(block_v, out_hbm.at[pl.ds(base, _CHUNK_ROWS)])


def _build_adj(edges):
    k = pl.kernel(
        _adj_body,
        out_type=jax.ShapeDtypeStruct((_N, _N), jnp.float32),
        mesh=plsc.VectorSubcoreMesh(core_axis_name="c", subcore_axis_name="s"),
        compiler_params=pltpu.CompilerParams(needs_layout_passes=False,
                                             use_tc_tiling_on_sc=True),
        scratch_types=[
            pltpu.VMEM((_E,), jnp.int32),
            pltpu.VMEM((_E,), jnp.int32),
            pltpu.VMEM((_CHUNK_ROWS, _N), jnp.float32),
        ],
    )
    return k(edges)


def _deg_kernel(adj_ref, o_ref):
    deg = jnp.sum(adj_ref[...], axis=1)
    o_ref[0, 0, :] = lax.rsqrt(deg)


def _compute_dinv(adj):
    out = pl.pallas_call(
        _deg_kernel,
        grid=(_N // 128,),
        in_specs=[pl.BlockSpec((128, _N), lambda i: (i, 0))],
        out_specs=pl.BlockSpec((1, 1, 128), lambda i: (i, 0, 0)),
        out_shape=jax.ShapeDtypeStruct((_N // 128, 1, 128), jnp.float32),
    )(adj)
    return out.reshape(_N)


def _scale_kernel(adj_ref, dcol_ref, drow_ref, o_ref, r_ref):
    v = adj_ref[...] * dcol_ref[:, 0:1] * drow_ref[...]
    o_ref[...] = v.astype(jnp.bfloat16)
    r_ref[0, 0, :] = jnp.sum(v, axis=1)


def _normalize_adj(adj, dinv):
    dcol = jnp.broadcast_to(dinv.reshape(_N, 1), (_N, 128))
    drow = dinv.reshape(1, _N)
    a, rowa = pl.pallas_call(
        _scale_kernel,
        grid=(_N // 128,),
        in_specs=[
            pl.BlockSpec((128, _N), lambda i: (i, 0)),
            pl.BlockSpec((128, 128), lambda i: (i, 0)),
            pl.BlockSpec((1, _N), lambda i: (0, 0)),
        ],
        out_specs=[
            pl.BlockSpec((128, _N), lambda i: (i, 0)),
            pl.BlockSpec((1, 1, 128), lambda i: (i, 0, 0)),
        ],
        out_shape=[
            jax.ShapeDtypeStruct((_N, _N), jnp.bfloat16),
            jax.ShapeDtypeStruct((_N // 128, 1, 128), jnp.float32),
        ],
    )(adj, dcol, drow)
    return a, rowa.reshape(_N)


def _sq_kernel(arow_ref, afull_ref, o_ref):
    v = jnp.dot(arow_ref[...], afull_ref[...],
                preferred_element_type=jnp.float32)
    o_ref[...] = v.astype(jnp.bfloat16)


def _square_adj(a, bm=512):
    return pl.pallas_call(
        _sq_kernel,
        grid=(_N // bm,),
        in_specs=[
            pl.BlockSpec((bm, _N), lambda i: (i, 0)),
            pl.BlockSpec((_N, _N), lambda i: (0, 0)),
        ],
        out_specs=pl.BlockSpec((bm, _N), lambda i: (i, 0)),
        out_shape=jax.ShapeDtypeStruct((_N, _N), jnp.bfloat16),
        compiler_params=pltpu.CompilerParams(
            dimension_semantics=("arbitrary",)),
    )(a, a)


def _xw_kernel(x_ref, w_ref, o_ref):
    v = jnp.dot(x_ref[...], w_ref[...], preferred_element_type=jnp.float32)
    o_ref[...] = v.astype(jnp.bfloat16)


def _dense(x, w, bm=1024):
    m, kdim = x.shape
    _, n = w.shape
    return pl.pallas_call(
        _xw_kernel,
        grid=(m // bm,),
        in_specs=[
            pl.BlockSpec((bm, kdim), lambda i: (i, 0)),
            pl.BlockSpec((kdim, n), lambda i: (0, 0)),
        ],
        out_specs=pl.BlockSpec((bm, n), lambda i: (i, 0)),
        out_shape=jax.ShapeDtypeStruct((m, n), jnp.bfloat16),
        compiler_params=pltpu.CompilerParams(
            dimension_semantics=("arbitrary",)),
    )(x, w)


def _gm_kernel(g_ref, w_ref, o_ref):
    v = jnp.dot(g_ref[...].astype(jnp.bfloat16), w_ref[...],
                preferred_element_type=jnp.float32)
    o_ref[...] = v


def _global_matmul(gf, w0c):
    return pl.pallas_call(
        _gm_kernel,
        out_shape=jax.ShapeDtypeStruct((_B, w0c.shape[1]), jnp.float32),
    )(gf, w0c)


_DNT = (((0,), (0,)), ((), ()))


def _l0_kernel(lft_ref, bvt_ref, gm_ref, wb_ref, wa_ref, o_ref):
    xt = lft_ref[0].astype(jnp.bfloat16)
    v = lax.dot_general(xt, wb_ref[...], _DNT,
                        preferred_element_type=jnp.float32)
    v += lax.dot_general(bvt_ref[:, pl.program_id(0), :], wa_ref[...], _DNT,
                         preferred_element_type=jnp.float32)
    v += gm_ref[0]
    o_ref[0] = v.astype(jnp.bfloat16)


def _layer0(lft, bvt, gm, w0b, w0a, bm=512):
    kdim = lft.shape[1]
    n = w0b.shape[1]
    return pl.pallas_call(
        _l0_kernel,
        grid=(_B, _N // bm),
        in_specs=[
            pl.BlockSpec((1, kdim, bm), lambda b, m: (b, 0, m)),
            pl.BlockSpec((3, _B, bm), lambda b, m: (0, 0, m)),
            pl.BlockSpec((1, 1, n), lambda b, m: (b, 0, 0)),
            pl.BlockSpec((kdim, n), lambda b, m: (0, 0)),
            pl.BlockSpec((3, n), lambda b, m: (0, 0)),
        ],
        out_specs=pl.BlockSpec((1, bm, n), lambda b, m: (b, m, 0)),
        out_shape=jax.ShapeDtypeStruct((_B, _N, n), jnp.bfloat16),
        compiler_params=pltpu.CompilerParams(
            dimension_semantics=("arbitrary", "arbitrary"),
            fuse_transposed_lhs_in_matmul=True),
    )(lft, bvt, gm, w0b, w0a)


def _leaky(v):
    return jnp.where(v >= 0, v, 0.01 * v)


def _pair_kernel(a2_ref, h_ref, rv_ref, bw_ref, b_ref, o_ref):
    v = jnp.dot(a2_ref[...], h_ref[0], preferred_element_type=jnp.float32)
    v += rv_ref[:, 0:1] * bw_ref[...]
    v += b_ref[...]
    o_ref[0] = _leaky(v)


def _pairdense_kernel(a2_ref, h_ref, rv_ref, bw_ref, b_ref, w_ref, o_ref):
    v = jnp.dot(a2_ref[...], h_ref[0], preferred_element_type=jnp.float32)
    v += rv_ref[:, 0:1] * bw_ref[...]
    v += b_ref[...]
    v = _leaky(v).astype(jnp.bfloat16)
    o_ref[0] = jnp.dot(v, w_ref[...],
                       preferred_element_type=jnp.float32).astype(jnp.bfloat16)


def _pair(a2, h, rv, bw, b, w=None):
    n = h.shape[-1]
    args = [a2, h, rv, bw, b]
    specs = [
        pl.BlockSpec((_N, _N), lambda i: (0, 0)),
        pl.BlockSpec((1, _N, n), lambda i: (i, 0, 0)),
        pl.BlockSpec((_N, 128), lambda i: (0, 0)),
        pl.BlockSpec((1, n), lambda i: (0, 0)),
        pl.BlockSpec((1, n), lambda i: (0, 0)),
    ]
    if w is None:
        body, nout, odt = _pair_kernel, n, jnp.float32
    else:
        args.append(w)
        specs.append(pl.BlockSpec((n, w.shape[1]), lambda i: (0, 0)))
        body, nout, odt = _pairdense_kernel, w.shape[1], jnp.bfloat16
    return pl.pallas_call(
        body,
        grid=(_B,),
        in_specs=specs,
        out_specs=pl.BlockSpec((1, _N, nout), lambda i: (i, 0, 0)),
        out_shape=jax.ShapeDtypeStruct((_B, _N, nout), odt),
        compiler_params=pltpu.CompilerParams(
            dimension_semantics=("arbitrary",)),
    )(*args)


def _dec_kernel(f_ref, w_ref, b_ref, o_ref):
    v = jnp.dot(f_ref[...], w_ref[...], preferred_element_type=jnp.float32)
    o_ref[...] = jnp.tanh(v + b_ref[...]) * 0.1


def _decoder(feats, w_dec, b_dec, bn=512):
    m, kdim = feats.shape
    return pl.pallas_call(
        _dec_kernel,
        grid=(kdim // bn,),
        in_specs=[
            pl.BlockSpec((m, kdim), lambda i: (0, 0)),
            pl.BlockSpec((kdim, bn), lambda i: (0, i)),
            pl.BlockSpec((1, bn), lambda i: (0, i)),
        ],
        out_specs=pl.BlockSpec((m, bn), lambda i: (0, i)),
        out_shape=jax.ShapeDtypeStruct((m, kdim), jnp.float32),
        compiler_params=pltpu.CompilerParams(
            dimension_semantics=("arbitrary",)),
    )(feats, w_dec, b_dec)


def _pad2(w, rows, cols):
    r, c = w.shape
    return jnp.pad(w, ((0, rows - r), (0, cols - c)))


def kernel(batch_vertices, local_features, global_features, edges,
           W0, b0, W1, b1, W2, b2, W3, b3, W4, b4, W5, b5, W_dec, b_dec):
    bf = jnp.bfloat16
    adj = _build_adj(edges)
    dinv = _compute_dinv(adj)
    a, rowa = _normalize_adj(adj, dinv)
    rv = jnp.broadcast_to(rowa.reshape(_N, 1), (_N, 128))
    a2 = _square_adj(a)

    w01aug = _dense(jnp.concatenate([W0, b0[None]], 0).astype(bf),
                    W1.astype(bf), bm=W0.shape[0] + 1)
    w23aug = _dense(jnp.concatenate([W2, b2[None]], 0).astype(bf),
                    W3.astype(bf), bm=513)
    w45aug = _dense(jnp.concatenate([W4, b4[None]], 0).astype(bf),
                    W5.astype(bf), bm=257)

    nf = local_features.shape[-1]
    w01a = w01aug[:3].astype(jnp.float32)
    w01b = w01aug[3:3 + nf]
    w01c = w01aug[3 + nf:3 + nf + _DIMS[1]]
    bw1 = w01aug[-1:].astype(jnp.float32)
    w23 = w23aug[:512]
    bw3 = w23aug[512:513].astype(jnp.float32)
    w45 = _pad2(w45aug[:256], 256, 128)
    bw5 = _pad2(w45aug[256:257].astype(jnp.float32), 1, 128)
    b5p = _pad2(b5.reshape(1, -1), 1, 128)

    gm = _global_matmul(global_features, w01c).reshape(_B, 1, _DIMS[1])
    lft = jnp.transpose(local_features, (0, 2, 1))
    bvt = jnp.transpose(batch_vertices, (2, 0, 1))
    h1 = _layer0(lft, bvt, gm, w01b, w01a)
    h2 = _pair(a2, h1, rv, bw1, b1.reshape(1, -1), w=w23)
    h3 = _pair(a2, h2, rv, bw3, b3.reshape(1, -1), w=w45)
    x6 = _pair(a2, h3, rv, bw5, b5p)

    feats = x6[..., :3].reshape(_B, _N * 3)
    out = _decoder(feats, W_dec, b_dec.reshape(1, _N * 3))
    return out.reshape(_B, _N, 3)


# speedup vs baseline: 1.6140x; 1.0254x over previous
import functools

import jax
import jax.numpy as jnp
from jax import lax
from jax.experimental import pallas as pl
from jax.experimental.pallas import tpu as pltpu
from jax.experimental.pallas import tpu_sc as plsc

_N = 2048
_B = 8
_E = 12288
_NTILES = 32
_CHUNK_ROWS = 32
_LANES = 16

_DIMS = [1536, 512, 512, 256, 256, 128, 128]


_POS = None


def _pmax(x, pos):
    for sh in (1, 2, 4, 8):
        sx = x.at[jnp.maximum(pos - sh, 0)].get(mode="promise_in_bounds")
        x = jnp.maximum(x, jnp.where(pos >= sh, sx, 0))
    return x


def _psum(x, pos):
    for sh in (1, 2, 4, 8):
        sx = x.at[jnp.maximum(pos - sh, 0)].get(mode="promise_in_bounds")
        x = x + jnp.where(pos >= sh, sx, 0)
    return x


def _adj_body(edges_hbm, out_hbm, src_v, dst_v, side_v, block_v):
    wid = lax.axis_index("s") * 2 + lax.axis_index("c")
    pltpu.sync_copy(edges_hbm.at[0], src_v)
    pltpu.sync_copy(edges_hbm.at[1], dst_v)

    big = jnp.int32(2147483647)
    pos = lax.iota(jnp.int32, _LANES)
    ones = jnp.ones((_LANES,), jnp.float32)
    base = wid * 64

    def zero_block():
        def zb(i, carry):
            block_v[i >> 7, pl.ds((i & 127) * _LANES, _LANES)] = (
                jnp.zeros((_LANES,), jnp.float32))
            return carry
        lax.fori_loop(0, _CHUNK_ROWS * _N // _LANES, zb, 0, unroll=8)

    def gat(x, idx):
        return x.at[idx].get(mode="promise_in_bounds")

    zero_block()

    def edge_body(i, off):
        d = dst_v[pl.ds(i * _LANES, _LANES)]
        s = src_v[pl.ds(i * _LANES, _LANES)]
        valid = (d >= base) & (d < base + 64)

        def hit(off):
            local = (d - base) * _N + s
            key = jnp.where(valid, local, big)
            skey = jnp.sort(key)
            prev = gat(skey, jnp.maximum(pos - 1, 0))
            nxt = gat(skey, jnp.minimum(pos + 1, _LANES - 1))
            is_start = (skey != prev) | (pos == 0)
            is_end = ((skey != nxt) | (pos == _LANES - 1)) & (skey != big)
            run_start = _pmax(jnp.where(is_start, pos, 0), pos)
            cnt = pos - run_start + 1
            lo = is_end & (skey < _CHUNK_ROWS * _N)
            hi = is_end & (skey >= _CHUNK_ROWS * _N) & (skey != big)
            rows = jnp.where(lo, lax.shift_right_logical(skey, 11), 0)
            cols = jnp.where(lo, jnp.bitwise_and(skey, _N - 1), 0)
            plsc.addupdate_scatter(block_v, [rows, cols],
                                   cnt.astype(jnp.float32), mask=lo)
            packed = jnp.bitwise_or(
                lax.shift_left(skey - _CHUNK_ROWS * _N, 5), cnt)
            plsc.store_compressed(side_v.at[pl.ds(off, _LANES)], packed,
                                  mask=hi)
            return off + jnp.sum(hi.astype(jnp.int32))

        return lax.cond(jnp.any(valid), hit, lambda o: o, off)

    nhi = lax.fori_loop(0, _E // _LANES, edge_body, jnp.int32(0))

    for j in range(2):
        r = pos + j * _LANES
        plsc.addupdate_scatter(block_v, [r, base + r], ones)
    pltpu.sync_copy(block_v, out_hbm.at[pl.ds(base, _CHUNK_ROWS)])

    zero_block()

    def side_body(i, carry):
        rem = nhi - i * _LANES
        lane_ok = pos < rem
        p = side_v[pl.ds(i * _LANES, _LANES)]
        kp = jnp.where(lane_ok, p, big)
        skp = jnp.sort(kp)
        kkey = lax.shift_right_logical(skp, 5)
        cntv = jnp.bitwise_and(skp, 31)
        prev = gat(kkey, jnp.maximum(pos - 1, 0))
        nxt = gat(kkey, jnp.minimum(pos + 1, _LANES - 1))
        is_start = (kkey != prev) | (pos == 0)
        is_end = (kkey != nxt) | (pos == _LANES - 1)
        cs = _psum(cntv, pos)
        run_start = _pmax(jnp.where(is_start, pos, 0), pos)
        csprev = gat(cs, jnp.maximum(run_start - 1, 0))
        total = cs - jnp.where(run_start > 0, csprev, 0)
        vend = is_end & (kkey < _CHUNK_ROWS * _N)
        rows = jnp.where(vend, lax.shift_right_logical(kkey, 11), 0)
        cols = jnp.where(vend, jnp.bitwise_and(kkey, _N - 1), 0)
        plsc.addupdate_scatter(block_v, [rows, cols],
                               total.astype(jnp.float32), mask=vend)
        return carry

    nb = (nhi + _LANES - 1) // _LANES
    lax.fori_loop(0, nb, side_body, 0)

    for j in range(2):
        r = pos + j * _LANES
        plsc.addupdate_scatter(block_v, [r, base + _CHUNK_ROWS + r], ones)
    pltpu.sync_copy(block_v,
                    out_hbm.at[pl.ds(base + _CHUNK_ROWS, _CHUNK_ROWS)])


def _build_adj(edges):
    k = pl.kernel(
        _adj_body,
        out_type=jax.ShapeDtypeStruct((_N, _N), jnp.float32),
        mesh=plsc.VectorSubcoreMesh(core_axis_name="c", subcore_axis_name="s"),
        compiler_params=pltpu.CompilerParams(needs_layout_passes=False,
                                             use_tc_tiling_on_sc=True),
        scratch_types=[
            pltpu.VMEM((_E,), jnp.int32),
            pltpu.VMEM((_E,), jnp.int32),
            pltpu.VMEM((_E + _LANES,), jnp.int32),
            pltpu.VMEM((_CHUNK_ROWS, _N), jnp.float32),
        ],
    )
    return k(edges)


def _deg_kernel(adj_ref, o_ref):
    deg = jnp.sum(adj_ref[...], axis=1)
    o_ref[0, 0, :] = lax.rsqrt(deg)


def _compute_dinv(adj):
    out = pl.pallas_call(
        _deg_kernel,
        grid=(_N // 128,),
        in_specs=[pl.BlockSpec((128, _N), lambda i: (i, 0))],
        out_specs=pl.BlockSpec((1, 1, 128), lambda i: (i, 0, 0)),
        out_shape=jax.ShapeDtypeStruct((_N // 128, 1, 128), jnp.float32),
    )(adj)
    return out.reshape(_N)


def _scale_kernel(adj_ref, dcol_ref, drow_ref, o_ref, r_ref):
    v = adj_ref[...] * dcol_ref[:, 0:1] * drow_ref[...]
    o_ref[...] = v.astype(jnp.bfloat16)
    r_ref[0, 0, :] = jnp.sum(v, axis=1)


def _normalize_adj(adj, dinv):
    dcol = jnp.broadcast_to(dinv.reshape(_N, 1), (_N, 128))
    drow = dinv.reshape(1, _N)
    a, rowa = pl.pallas_call(
        _scale_kernel,
        grid=(_N // 128,),
        in_specs=[
            pl.BlockSpec((128, _N), lambda i: (i, 0)),
            pl.BlockSpec((128, 128), lambda i: (i, 0)),
            pl.BlockSpec((1, _N), lambda i: (0, 0)),
        ],
        out_specs=[
            pl.BlockSpec((128, _N), lambda i: (i, 0)),
            pl.BlockSpec((1, 1, 128), lambda i: (i, 0, 0)),
        ],
        out_shape=[
            jax.ShapeDtypeStruct((_N, _N), jnp.bfloat16),
            jax.ShapeDtypeStruct((_N // 128, 1, 128), jnp.float32),
        ],
    )(adj, dcol, drow)
    return a, rowa.reshape(_N)


def _sq_kernel(arow_ref, afull_ref, o_ref):
    v = jnp.dot(arow_ref[...], afull_ref[...],
                preferred_element_type=jnp.float32)
    o_ref[...] = v.astype(jnp.bfloat16)


def _square_adj(a, bm=512):
    return pl.pallas_call(
        _sq_kernel,
        grid=(_N // bm,),
        in_specs=[
            pl.BlockSpec((bm, _N), lambda i: (i, 0)),
            pl.BlockSpec((_N, _N), lambda i: (0, 0)),
        ],
        out_specs=pl.BlockSpec((bm, _N), lambda i: (i, 0)),
        out_shape=jax.ShapeDtypeStruct((_N, _N), jnp.bfloat16),
        compiler_params=pltpu.CompilerParams(
            dimension_semantics=("arbitrary",)),
    )(a, a)


def _xw_kernel(x_ref, w_ref, o_ref):
    v = jnp.dot(x_ref[...], w_ref[...], preferred_element_type=jnp.float32)
    o_ref[...] = v.astype(jnp.bfloat16)


def _dense(x, w, bm=1024):
    m, kdim = x.shape
    _, n = w.shape
    return pl.pallas_call(
        _xw_kernel,
        grid=(m // bm,),
        in_specs=[
            pl.BlockSpec((bm, kdim), lambda i: (i, 0)),
            pl.BlockSpec((kdim, n), lambda i: (0, 0)),
        ],
        out_specs=pl.BlockSpec((bm, n), lambda i: (i, 0)),
        out_shape=jax.ShapeDtypeStruct((m, n), jnp.bfloat16),
        compiler_params=pltpu.CompilerParams(
            dimension_semantics=("arbitrary",)),
    )(x, w)


def _gm_kernel(g_ref, w_ref, o_ref):
    v = jnp.dot(g_ref[...].astype(jnp.bfloat16), w_ref[...],
                preferred_element_type=jnp.float32)
    o_ref[...] = v


def _global_matmul(gf, w0c):
    return pl.pallas_call(
        _gm_kernel,
        out_shape=jax.ShapeDtypeStruct((_B, w0c.shape[1]), jnp.float32),
    )(gf, w0c)


_DNT = (((0,), (0,)), ((), ()))


def _l0_kernel(lft_ref, bvt_ref, gm_ref, wb_ref, wa_ref, o_ref):
    xt = lft_ref[0].astype(jnp.bfloat16)
    v = lax.dot_general(xt, wb_ref[...], _DNT,
                        preferred_element_type=jnp.float32)
    v += lax.dot_general(bvt_ref[:, pl.program_id(0), :], wa_ref[...], _DNT,
                         preferred_element_type=jnp.float32)
    v += gm_ref[0]
    o_ref[0] = v.astype(jnp.bfloat16)


def _layer0(lft, bvt, gm, w0b, w0a, bm=512):
    kdim = lft.shape[1]
    n = w0b.shape[1]
    return pl.pallas_call(
        _l0_kernel,
        grid=(_B, _N // bm),
        in_specs=[
            pl.BlockSpec((1, kdim, bm), lambda b, m: (b, 0, m)),
            pl.BlockSpec((3, _B, bm), lambda b, m: (0, 0, m)),
            pl.BlockSpec((1, 1, n), lambda b, m: (b, 0, 0)),
            pl.BlockSpec((kdim, n), lambda b, m: (0, 0)),
            pl.BlockSpec((3, n), lambda b, m: (0, 0)),
        ],
        out_specs=pl.BlockSpec((1, bm, n), lambda b, m: (b, m, 0)),
        out_shape=jax.ShapeDtypeStruct((_B, _N, n), jnp.bfloat16),
        compiler_params=pltpu.CompilerParams(
            dimension_semantics=("arbitrary", "arbitrary"),
            fuse_transposed_lhs_in_matmul=True),
    )(lft, bvt, gm, w0b, w0a)


def _leaky(v):
    return jnp.where(v >= 0, v, 0.01 * v)


def _pair_kernel(a2_ref, h_ref, rv_ref, bw_ref, b_ref, o_ref):
    v = jnp.dot(a2_ref[...], h_ref[0], preferred_element_type=jnp.float32)
    v += rv_ref[:, 0:1] * bw_ref[...]
    v += b_ref[...]
    o_ref[0] = _leaky(v)


def _pairdense_kernel(a2_ref, h_ref, rv_ref, bw_ref, b_ref, w_ref, o_ref):
    v = jnp.dot(a2_ref[...], h_ref[0], preferred_element_type=jnp.float32)
    v += rv_ref[:, 0:1] * bw_ref[...]
    v += b_ref[...]
    v = _leaky(v).astype(jnp.bfloat16)
    o_ref[0] = jnp.dot(v, w_ref[...],
                       preferred_element_type=jnp.float32).astype(jnp.bfloat16)


def _pair(a2, h, rv, bw, b, w=None):
    n = h.shape[-1]
    args = [a2, h, rv, bw, b]
    specs = [
        pl.BlockSpec((_N, _N), lambda i: (0, 0)),
        pl.BlockSpec((1, _N, n), lambda i: (i, 0, 0)),
        pl.BlockSpec((_N, 128), lambda i: (0, 0)),
        pl.BlockSpec((1, n), lambda i: (0, 0)),
        pl.BlockSpec((1, n), lambda i: (0, 0)),
    ]
    if w is None:
        body, nout, odt = _pair_kernel, n, jnp.float32
    else:
        args.append(w)
        specs.append(pl.BlockSpec((n, w.shape[1]), lambda i: (0, 0)))
        body, nout, odt = _pairdense_kernel, w.shape[1], jnp.bfloat16
    return pl.pallas_call(
        body,
        grid=(_B,),
        in_specs=specs,
        out_specs=pl.BlockSpec((1, _N, nout), lambda i: (i, 0, 0)),
        out_shape=jax.ShapeDtypeStruct((_B, _N, nout), odt),
        compiler_params=pltpu.CompilerParams(
            dimension_semantics=("arbitrary",)),
    )(*args)


def _dec_kernel(f_ref, w_ref, b_ref, o_ref):
    v = jnp.dot(f_ref[...], w_ref[...], preferred_element_type=jnp.float32)
    o_ref[...] = jnp.tanh(v + b_ref[...]) * 0.1


def _decoder(feats, w_dec, b_dec, bn=512):
    m, kdim = feats.shape
    return pl.pallas_call(
        _dec_kernel,
        grid=(kdim // bn,),
        in_specs=[
            pl.BlockSpec((m, kdim), lambda i: (0, 0)),
            pl.BlockSpec((kdim, bn), lambda i: (0, i)),
            pl.BlockSpec((1, bn), lambda i: (0, i)),
        ],
        out_specs=pl.BlockSpec((m, bn), lambda i: (0, i)),
        out_shape=jax.ShapeDtypeStruct((m, kdim), jnp.float32),
        compiler_params=pltpu.CompilerParams(
            dimension_semantics=("arbitrary",)),
    )(feats, w_dec, b_dec)


def _pad2(w, rows, cols):
    r, c = w.shape
    return jnp.pad(w, ((0, rows - r), (0, cols - c)))


def kernel(batch_vertices, local_features, global_features, edges,
           W0, b0, W1, b1, W2, b2, W3, b3, W4, b4, W5, b5, W_dec, b_dec):
    bf = jnp.bfloat16
    adj = _build_adj(edges)
    dinv = _compute_dinv(adj)
    a, rowa = _normalize_adj(adj, dinv)
    rv = jnp.broadcast_to(rowa.reshape(_N, 1), (_N, 128))
    a2 = _square_adj(a)

    w01aug = _dense(jnp.concatenate([W0, b0[None]], 0).astype(bf),
                    W1.astype(bf), bm=W0.shape[0] + 1)
    w23aug = _dense(jnp.concatenate([W2, b2[None]], 0).astype(bf),
                    W3.astype(bf), bm=513)
    w45aug = _dense(jnp.concatenate([W4, b4[None]], 0).astype(bf),
                    W5.astype(bf), bm=257)

    nf = local_features.shape[-1]
    w01a = w01aug[:3].astype(jnp.float32)
    w01b = w01aug[3:3 + nf]
    w01c = w01aug[3 + nf:3 + nf + _DIMS[1]]
    bw1 = w01aug[-1:].astype(jnp.float32)
    w23 = w23aug[:512]
    bw3 = w23aug[512:513].astype(jnp.float32)
    w45 = _pad2(w45aug[:256], 256, 128)
    bw5 = _pad2(w45aug[256:257].astype(jnp.float32), 1, 128)
    b5p = _pad2(b5.reshape(1, -1), 1, 128)

    gm = _global_matmul(global_features, w01c).reshape(_B, 1, _DIMS[1])
    lft = jnp.transpose(local_features, (0, 2, 1))
    bvt = jnp.transpose(batch_vertices, (2, 0, 1))
    h1 = _layer0(lft, bvt, gm, w01b, w01a)
    h2 = _pair(a2, h1, rv, bw1, b1.reshape(1, -1), w=w23)
    h3 = _pair(a2, h2, rv, bw3, b3.reshape(1, -1), w=w45)
    x6 = _pair(a2, h3, rv, bw5, b5p)

    feats = x6[..., :3].reshape(_B, _N * 3)
    out = _decoder(feats, W_dec, b_dec.reshape(1, _N * 3))
    return out.reshape(_B, _N, 3)
